# Initial kernel scaffold; baseline (speedup 1.0000x reference)
import jax
import jax.numpy as jnp
from jax.experimental import pallas as pl


def kernel(x, Wg, W1, b1, W2, b2):
    raise NotImplementedError("write your pallas kernel here")


# speedup vs baseline: 1.2697x; 1.2697x over previous
import functools

import jax
import jax.numpy as jnp
from jax.experimental import pallas as pl
from jax.experimental.pallas import tpu as pltpu

E = 64
D = 1024
H = 4096
TOPK = 2
BALANCE_COEFF = 0.01
N_TOK = 32
HT = 512
NH = H // HT


def _gate_kernel(x_ref, wg_ref, g_ref, meta_ref, aux_ref):
    x = x_ref[...]
    wg = wg_ref[...]
    logits = jax.lax.dot_general(
        x, wg, (((1,), (1,)), ((), ())), preferred_element_type=jnp.float32
    )
    lane = jax.lax.broadcasted_iota(jnp.int32, (N_TOK, E), 1)
    m1 = jnp.max(logits, axis=1, keepdims=True)
    idx1 = jnp.min(jnp.where(logits == m1, lane, E), axis=1, keepdims=True)
    masked = jnp.where(lane == idx1, jnp.float32(-1e30), logits)
    m2 = jnp.max(masked, axis=1, keepdims=True)
    idx2 = jnp.min(jnp.where(masked == m2, lane, E), axis=1, keepdims=True)
    s = jnp.exp(m2 - m1)
    g1 = 1.0 / (1.0 + s)
    g2 = s / (1.0 + s)
    sel1 = lane == idx1
    sel2 = lane == idx2
    g_ref[...] = jnp.where(sel1, g1, 0.0) + jnp.where(sel2, g2, 0.0)

    counts = jnp.sum(
        jnp.where(sel1, 1.0, 0.0) + jnp.where(sel2, 1.0, 0.0),
        axis=0, keepdims=True)
    load = counts * (1.0 / (N_TOK * TOPK))
    aux_ref[...] = jnp.reshape(
        BALANCE_COEFF * (E * jnp.sum(load * load)), (1, 1))

    sel = counts > 0.0
    self32 = jnp.where(sel, 1.0, 0.0)
    e_row = jax.lax.broadcasted_iota(jnp.int32, (E, E), 0)
    e_col = jax.lax.broadcasted_iota(jnp.int32, (E, E), 1)
    tri = jnp.where(e_row <= e_col, 1.0, 0.0)
    csum = jax.lax.dot_general(
        self32, tri, (((1,), (0,)), ((), ())),
        preferred_element_type=jnp.float32)
    rank = csum - self32
    nf = jnp.sum(self32)

    selb = jnp.broadcast_to(sel, (E, E))
    rankb = jnp.broadcast_to(rank, (E, E))
    match = selb & (rankb == e_row.astype(jnp.float32))
    ids_col = jnp.sum(jnp.where(match, e_col.astype(jnp.float32), 0.0),
                      axis=1, keepdims=True)
    p_col = jax.lax.broadcasted_iota(jnp.int32, (E, 1), 0).astype(jnp.float32)
    last_id = jnp.sum(jnp.where(p_col == nf - 1.0, ids_col, 0.0))
    ids_pad = jnp.where(p_col < nf, ids_col, last_id)

    p_row = jax.lax.broadcasted_iota(jnp.int32, (E, 128), 0)
    j_col = jax.lax.broadcasted_iota(jnp.int32, (E, 128), 1)
    idsb = jnp.broadcast_to(ids_pad, (E, 128))
    ids_row = jnp.sum(jnp.where(p_row == j_col, idsb, 0.0),
                      axis=0, keepdims=True)
    j1 = jax.lax.broadcasted_iota(jnp.int32, (1, 128), 1)
    meta = jnp.where(j1 == E, nf, ids_row)
    meta_ref[...] = meta.astype(jnp.int32)


def _ffn_kernel(meta_ref, x_ref, g_ref, w1_ref, b1_ref, w2_ref, b2_ref,
                out_ref):
    i = pl.program_id(0)
    h = pl.program_id(1)

    @pl.when((i == 0) & (h == 0))
    def _init():
        out_ref[...] = jnp.zeros_like(out_ref)

    n = meta_ref[0, E]

    @pl.when(i < n)
    def _body():
        e = meta_ref[0, i]
        lane = jax.lax.broadcasted_iota(jnp.int32, (N_TOK, E), 1)
        w = jnp.sum(jnp.where(lane == e, g_ref[...], 0.0),
                    axis=1, keepdims=True)
        hp = jax.lax.dot_general(
            x_ref[...], w1_ref[0], (((1,), (1,)), ((), ())),
            preferred_element_type=jnp.float32)
        hp = hp + b1_ref[0, :, pl.ds(h * HT, HT)]
        hp = hp * jax.nn.sigmoid(hp)
        yp = jax.lax.dot_general(
            hp, w2_ref[0], (((1,), (1,)), ((), ())),
            preferred_element_type=jnp.float32)
        yp = yp + jnp.where(h == 0, 1.0, 0.0) * b2_ref[0]
        out_ref[...] += w * yp


@jax.jit
def kernel(x, Wg, W1, b1, W2, b2):
    orig_shape = x.shape
    x_flat = x.reshape(-1, x.shape[-1])

    G, meta, aux = pl.pallas_call(
        _gate_kernel,
        out_shape=(
            jax.ShapeDtypeStruct((N_TOK, E), jnp.float32),
            jax.ShapeDtypeStruct((1, 128), jnp.int32),
            jax.ShapeDtypeStruct((1, 1), jnp.float32),
        ),
    )(x_flat, Wg)

    b1r = b1.reshape(E, 1, H)
    b2r = b2.reshape(E, 1, D)

    def _w1_map(i, h, m):
        pad = i >= m[0, E]
        return (m[0, i], jnp.where(pad, NH - 1, h), 0)

    def _w2_map(i, h, m):
        pad = i >= m[0, E]
        return (m[0, i], 0, jnp.where(pad, NH - 1, h))

    def _eb_map(i, h, m):
        return (m[0, i], 0, 0)

    def _const_map(i, h, m):
        return (0, 0)

    grid_spec = pltpu.PrefetchScalarGridSpec(
        num_scalar_prefetch=1,
        grid=(E, NH),
        in_specs=[
            pl.BlockSpec((N_TOK, D), _const_map),
            pl.BlockSpec((N_TOK, E), _const_map),
            pl.BlockSpec((1, HT, D), _w1_map),
            pl.BlockSpec((1, 1, H), _eb_map),
            pl.BlockSpec((1, D, HT), _w2_map),
            pl.BlockSpec((1, 1, D), _eb_map),
        ],
        out_specs=pl.BlockSpec((N_TOK, D), _const_map),
    )

    out = pl.pallas_call(
        _ffn_kernel,
        grid_spec=grid_spec,
        out_shape=jax.ShapeDtypeStruct((N_TOK, D), jnp.float32),
        compiler_params=pltpu.CompilerParams(
            dimension_semantics=("arbitrary", "arbitrary")),
    )(meta, x_flat, G, W1, b1r, W2, b2r)

    output = out.reshape(orig_shape)
    aux_loss = jnp.reshape(aux, ())
    return output, aux_loss


# speedup vs baseline: 1.6131x; 1.2704x over previous
import functools

import jax
import jax.numpy as jnp
from jax.experimental import pallas as pl
from jax.experimental.pallas import tpu as pltpu

E = 64
D = 1024
H = 4096
TOPK = 2
BALANCE_COEFF = 0.01
N_TOK = 32
HT = 1024
NH = H // HT


def _gate_kernel(x_ref, wg_ref, g_ref, meta_ref, aux_ref):
    x = x_ref[...]
    wg = wg_ref[...]
    logits = jax.lax.dot_general(
        x, wg, (((1,), (1,)), ((), ())), preferred_element_type=jnp.float32
    )
    lane = jax.lax.broadcasted_iota(jnp.int32, (N_TOK, E), 1)
    m1 = jnp.max(logits, axis=1, keepdims=True)
    idx1 = jnp.min(jnp.where(logits == m1, lane, E), axis=1, keepdims=True)
    masked = jnp.where(lane == idx1, jnp.float32(-1e30), logits)
    m2 = jnp.max(masked, axis=1, keepdims=True)
    idx2 = jnp.min(jnp.where(masked == m2, lane, E), axis=1, keepdims=True)
    s = jnp.exp(m2 - m1)
    g1 = 1.0 / (1.0 + s)
    g2 = s / (1.0 + s)
    sel1 = lane == idx1
    sel2 = lane == idx2
    g_ref[...] = jnp.where(sel1, g1, 0.0) + jnp.where(sel2, g2, 0.0)

    counts = jnp.sum(
        jnp.where(sel1, 1.0, 0.0) + jnp.where(sel2, 1.0, 0.0),
        axis=0, keepdims=True)
    load = counts * (1.0 / (N_TOK * TOPK))
    aux_ref[...] = jnp.reshape(
        BALANCE_COEFF * (E * jnp.sum(load * load)), (1, 1))

    sel = counts > 0.0
    self32 = jnp.where(sel, 1.0, 0.0)
    e_row = jax.lax.broadcasted_iota(jnp.int32, (E, E), 0)
    e_col = jax.lax.broadcasted_iota(jnp.int32, (E, E), 1)
    tri = jnp.where(e_row <= e_col, 1.0, 0.0)
    csum = jax.lax.dot_general(
        self32, tri, (((1,), (0,)), ((), ())),
        preferred_element_type=jnp.float32)
    rank = csum - self32
    nf = jnp.sum(self32)

    selb = jnp.broadcast_to(sel, (E, E))
    rankb = jnp.broadcast_to(rank, (E, E))
    match = selb & (rankb == e_row.astype(jnp.float32))
    ids_col = jnp.sum(jnp.where(match, e_col.astype(jnp.float32), 0.0),
                      axis=1, keepdims=True)
    p_col = jax.lax.broadcasted_iota(jnp.int32, (E, 1), 0).astype(jnp.float32)
    last_id = jnp.sum(jnp.where(p_col == nf - 1.0, ids_col, 0.0))
    ids_pad = jnp.where(p_col < nf, ids_col, last_id)

    p_row = jax.lax.broadcasted_iota(jnp.int32, (E, 128), 0)
    j_col = jax.lax.broadcasted_iota(jnp.int32, (E, 128), 1)
    idsb = jnp.broadcast_to(ids_pad, (E, 128))
    ids_row = jnp.sum(jnp.where(p_row == j_col, idsb, 0.0),
                      axis=0, keepdims=True)
    j1 = jax.lax.broadcasted_iota(jnp.int32, (1, 128), 1)
    meta = jnp.where(j1 == E, nf, ids_row)
    meta_ref[...] = meta.astype(jnp.int32)


def _ffn_kernel(meta_ref, x_ref, g_ref, w1_ref, b1_ref, w2_ref, b2_ref,
                out_ref):
    i = pl.program_id(0)
    h = pl.program_id(1)

    @pl.when((i == 0) & (h == 0))
    def _init():
        out_ref[...] = jnp.zeros_like(out_ref)

    n = meta_ref[0, E]

    @pl.when(i < n)
    def _body():
        e = meta_ref[0, i]
        lane = jax.lax.broadcasted_iota(jnp.int32, (N_TOK, E), 1)
        w = jnp.sum(jnp.where(lane == e, g_ref[...], 0.0),
                    axis=1, keepdims=True)
        hp = jax.lax.dot_general(
            x_ref[...], w1_ref[0], (((1,), (1,)), ((), ())),
            preferred_element_type=jnp.float32)
        hp = hp + b1_ref[0, :, pl.ds(h * HT, HT)]
        hp = hp * jax.nn.sigmoid(hp)
        yp = jax.lax.dot_general(
            hp, w2_ref[0], (((1,), (1,)), ((), ())),
            preferred_element_type=jnp.float32)
        yp = yp + jnp.where(h == 0, 1.0, 0.0) * b2_ref[0]
        out_ref[...] += w * yp


@jax.jit
def kernel(x, Wg, W1, b1, W2, b2):
    orig_shape = x.shape
    x_flat = x.reshape(-1, x.shape[-1])

    G, meta, aux = pl.pallas_call(
        _gate_kernel,
        out_shape=(
            jax.ShapeDtypeStruct((N_TOK, E), jnp.float32),
            jax.ShapeDtypeStruct((1, 128), jnp.int32),
            jax.ShapeDtypeStruct((1, 1), jnp.float32),
        ),
    )(x_flat, Wg)

    b1r = b1.reshape(E, 1, H)
    b2r = b2.reshape(E, 1, D)

    def _w1_map(i, h, m):
        pad = i >= m[0, E]
        return (m[0, i], jnp.where(pad, NH - 1, h), 0)

    def _w2_map(i, h, m):
        pad = i >= m[0, E]
        return (m[0, i], 0, jnp.where(pad, NH - 1, h))

    def _eb_map(i, h, m):
        return (m[0, i], 0, 0)

    def _const_map(i, h, m):
        return (0, 0)

    grid_spec = pltpu.PrefetchScalarGridSpec(
        num_scalar_prefetch=1,
        grid=(E, NH),
        in_specs=[
            pl.BlockSpec((N_TOK, D), _const_map),
            pl.BlockSpec((N_TOK, E), _const_map),
            pl.BlockSpec((1, HT, D), _w1_map),
            pl.BlockSpec((1, 1, H), _eb_map),
            pl.BlockSpec((1, D, HT), _w2_map),
            pl.BlockSpec((1, 1, D), _eb_map),
        ],
        out_specs=pl.BlockSpec((N_TOK, D), _const_map),
    )

    out = pl.pallas_call(
        _ffn_kernel,
        grid_spec=grid_spec,
        out_shape=jax.ShapeDtypeStruct((N_TOK, D), jnp.float32),
        compiler_params=pltpu.CompilerParams(
            dimension_semantics=("arbitrary", "arbitrary")),
    )(meta, x_flat, G, W1, b1r, W2, b2r)

    output = out.reshape(orig_shape)
    aux_loss = jnp.reshape(aux, ())
    return output, aux_loss


# speedup vs baseline: 1.6520x; 1.0242x over previous
import functools

import jax
import jax.numpy as jnp
from jax.experimental import pallas as pl
from jax.experimental.pallas import tpu as pltpu

E = 64
D = 1024
H = 4096
TOPK = 2
BALANCE_COEFF = 0.01
N_TOK = 32
HT = 2048
NH = H // HT


def _gate_kernel(x_ref, wg_ref, g_ref, meta_ref, aux_ref):
    x = x_ref[...]
    wg = wg_ref[...]
    logits = jax.lax.dot_general(
        x, wg, (((1,), (1,)), ((), ())), preferred_element_type=jnp.float32
    )
    lane = jax.lax.broadcasted_iota(jnp.int32, (N_TOK, E), 1)
    m1 = jnp.max(logits, axis=1, keepdims=True)
    idx1 = jnp.min(jnp.where(logits == m1, lane, E), axis=1, keepdims=True)
    masked = jnp.where(lane == idx1, jnp.float32(-1e30), logits)
    m2 = jnp.max(masked, axis=1, keepdims=True)
    idx2 = jnp.min(jnp.where(masked == m2, lane, E), axis=1, keepdims=True)
    s = jnp.exp(m2 - m1)
    g1 = 1.0 / (1.0 + s)
    g2 = s / (1.0 + s)
    sel1 = lane == idx1
    sel2 = lane == idx2
    g_ref[...] = jnp.where(sel1, g1, 0.0) + jnp.where(sel2, g2, 0.0)

    counts = jnp.sum(
        jnp.where(sel1, 1.0, 0.0) + jnp.where(sel2, 1.0, 0.0),
        axis=0, keepdims=True)
    load = counts * (1.0 / (N_TOK * TOPK))
    aux_ref[...] = jnp.reshape(
        BALANCE_COEFF * (E * jnp.sum(load * load)), (1, 1))

    sel = counts > 0.0
    self32 = jnp.where(sel, 1.0, 0.0)
    e_row = jax.lax.broadcasted_iota(jnp.int32, (E, E), 0)
    e_col = jax.lax.broadcasted_iota(jnp.int32, (E, E), 1)
    tri = jnp.where(e_row <= e_col, 1.0, 0.0)
    csum = jax.lax.dot_general(
        self32, tri, (((1,), (0,)), ((), ())),
        preferred_element_type=jnp.float32)
    rank = csum - self32
    nf = jnp.sum(self32)

    selb = jnp.broadcast_to(sel, (E, E))
    rankb = jnp.broadcast_to(rank, (E, E))
    match = selb & (rankb == e_row.astype(jnp.float32))
    ids_col = jnp.sum(jnp.where(match, e_col.astype(jnp.float32), 0.0),
                      axis=1, keepdims=True)
    p_col = jax.lax.broadcasted_iota(jnp.int32, (E, 1), 0).astype(jnp.float32)
    last_id = jnp.sum(jnp.where(p_col == nf - 1.0, ids_col, 0.0))
    ids_pad = jnp.where(p_col < nf, ids_col, last_id)

    p_row = jax.lax.broadcasted_iota(jnp.int32, (E, 128), 0)
    j_col = jax.lax.broadcasted_iota(jnp.int32, (E, 128), 1)
    idsb = jnp.broadcast_to(ids_pad, (E, 128))
    ids_row = jnp.sum(jnp.where(p_row == j_col, idsb, 0.0),
                      axis=0, keepdims=True)
    j1 = jax.lax.broadcasted_iota(jnp.int32, (1, 128), 1)
    meta = jnp.where(j1 == E, nf, ids_row)
    meta_ref[...] = meta.astype(jnp.int32)


def _ffn_kernel(meta_ref, x_ref, g_ref, w1_ref, b1_ref, w2_ref, b2_ref,
                out_ref):
    i = pl.program_id(0)
    h = pl.program_id(1)

    @pl.when((i == 0) & (h == 0))
    def _init():
        out_ref[...] = jnp.zeros_like(out_ref)

    n = meta_ref[0, E]

    @pl.when(i < n)
    def _body():
        e = meta_ref[0, i]
        lane = jax.lax.broadcasted_iota(jnp.int32, (N_TOK, E), 1)
        w = jnp.sum(jnp.where(lane == e, g_ref[...], 0.0),
                    axis=1, keepdims=True)
        hp = jax.lax.dot_general(
            x_ref[...], w1_ref[0], (((1,), (1,)), ((), ())),
            preferred_element_type=jnp.float32)
        hp = hp + b1_ref[0, :, pl.ds(h * HT, HT)]
        hp = hp * jax.nn.sigmoid(hp)
        yp = jax.lax.dot_general(
            hp, w2_ref[0], (((1,), (1,)), ((), ())),
            preferred_element_type=jnp.float32)
        yp = yp + jnp.where(h == 0, 1.0, 0.0) * b2_ref[0]
        out_ref[...] += w * yp


@jax.jit
def kernel(x, Wg, W1, b1, W2, b2):
    orig_shape = x.shape
    x_flat = x.reshape(-1, x.shape[-1])

    G, meta, aux = pl.pallas_call(
        _gate_kernel,
        out_shape=(
            jax.ShapeDtypeStruct((N_TOK, E), jnp.float32),
            jax.ShapeDtypeStruct((1, 128), jnp.int32),
            jax.ShapeDtypeStruct((1, 1), jnp.float32),
        ),
    )(x_flat, Wg)

    b1r = b1.reshape(E, 1, H)
    b2r = b2.reshape(E, 1, D)

    def _w1_map(i, h, m):
        pad = i >= m[0, E]
        return (m[0, i], jnp.where(pad, NH - 1, h), 0)

    def _w2_map(i, h, m):
        pad = i >= m[0, E]
        return (m[0, i], 0, jnp.where(pad, NH - 1, h))

    def _eb_map(i, h, m):
        return (m[0, i], 0, 0)

    def _const_map(i, h, m):
        return (0, 0)

    grid_spec = pltpu.PrefetchScalarGridSpec(
        num_scalar_prefetch=1,
        grid=(E, NH),
        in_specs=[
            pl.BlockSpec((N_TOK, D), _const_map),
            pl.BlockSpec((N_TOK, E), _const_map),
            pl.BlockSpec((1, HT, D), _w1_map),
            pl.BlockSpec((1, 1, H), _eb_map),
            pl.BlockSpec((1, D, HT), _w2_map),
            pl.BlockSpec((1, 1, D), _eb_map),
        ],
        out_specs=pl.BlockSpec((N_TOK, D), _const_map),
    )

    out = pl.pallas_call(
        _ffn_kernel,
        grid_spec=grid_spec,
        out_shape=jax.ShapeDtypeStruct((N_TOK, D), jnp.float32),
        compiler_params=pltpu.CompilerParams(
            dimension_semantics=("arbitrary", "arbitrary")),
    )(meta, x_flat, G, W1, b1r, W2, b2r)

    output = out.reshape(orig_shape)
    aux_loss = jnp.reshape(aux, ())
    return output, aux_loss
